# baseline (speedup 1.0000x reference)
import functools
import math

import jax
import jax.numpy as jnp
from jax.experimental import pallas as pl

_BIG = 3.0e38


def _batched_cluster(xs, noise_row, k, n_clusters):
    F = len(xs)
    N, C = xs[0].shape
    s = jnp.float32(math.sqrt(C))
    d2s = []
    dmaxs = []
    for x in xs:
        x2 = jnp.sum(x * x, axis=1, keepdims=True)
        dot = jnp.dot(x, x.T, preferred_element_type=jnp.float32)
        d2 = jnp.maximum((x2 + x2.T) - 2.0 * dot, 0.0)
        d2s.append(d2)
        dmaxs.append(jnp.max(d2))
    D = jnp.concatenate(d2s, axis=1)
    del d2s

    acc = jnp.zeros((1, F * N), jnp.float32)
    taken = jnp.zeros((1, F * N), jnp.float32)
    dmm = D
    for r in range(k):
        minv = jnp.min(dmm, axis=0, keepdims=True)
        t = jnp.sqrt(minv) / s
        eq = dmm == minv
        c = jnp.sum(eq.astype(jnp.float32), axis=0, keepdims=True)
        m = jnp.minimum(c, k - taken)
        acc = acc + m * (t * t)
        taken = taken + m
        if r < k - 1:
            dmm = jnp.where(eq, _BIG, dmm)
    density = jnp.exp(-(acc / k)) + noise_row

    dists = []
    for f in range(F):
        drow = density[:, f * N:(f + 1) * N]
        m = jnp.where(drow.T > drow, D[:, f * N:(f + 1) * N], dmaxs[f])
        dists.append(jnp.min(m, axis=0, keepdims=True))
    dist = jnp.sqrt(jnp.concatenate(dists, axis=1)) / s
    score = dist * density

    io0 = jax.lax.broadcasted_iota(jnp.int32, (N, N), 0)
    io1 = jax.lax.broadcasted_iota(jnp.int32, (N, N), 1)
    lower = io0 < io1
    idx_rows = []
    merged = []
    for f in range(F):
        Dv = D[:, f * N:(f + 1) * N]
        sr = score[:, f * N:(f + 1) * N]
        sc = sr.T
        beats = (sc > sr) | ((sc == sr) & lower)
        rank = jnp.sum(beats.astype(jnp.float32), axis=0, keepdims=True)
        rank_col = rank.T
        icen_col = rank_col < n_clusters
        masked = jnp.where(icen_col, Dv, _BIG)
        mv = jnp.min(masked, axis=0, keepdims=True)
        rc = jnp.where(icen_col, rank_col, jnp.float32(n_clusters))
        cl = jnp.min(jnp.where(masked == mv, rc, jnp.float32(n_clusters)),
                     axis=0, keepdims=True)
        idx_row = jnp.where(rank < n_clusters, rank, cl)
        assign = (jax.lax.broadcasted_iota(
            jnp.int32, (n_clusters, N), 0).astype(jnp.float32)
            == idx_row).astype(jnp.float32)
        counts = jnp.sum(assign, axis=1, keepdims=True)
        mrg = jnp.dot(assign, xs[f], preferred_element_type=jnp.float32)
        merged.append(mrg / (counts + 1e-6))
        idx_rows.append(idx_row)
    return idx_rows, merged


def _group(meta1, meta2, idx2_row, w, b):
    K1, C = meta1.shape
    K2 = meta2.shape[0]
    M = K1 + K2
    i_m = jax.lax.broadcasted_iota(jnp.int32, (1, M), 1).astype(jnp.float32)
    vals = jnp.concatenate(
        [jax.lax.broadcasted_iota(jnp.int32, (1, K2), 1).astype(jnp.float32),
         idx2_row], axis=1)
    rank = jnp.where(i_m < K2, 0.0, i_m - jnp.float32(K2 - 1))
    keys = vals * jnp.float32(K1 + 1) + rank
    pos = jnp.sum((keys < keys.T).astype(jnp.float32), axis=1, keepdims=True)
    perm = (jax.lax.broadcasted_iota(jnp.int32, (M, M), 0).astype(jnp.float32)
            == pos.T).astype(jnp.float32)
    combined = jnp.concatenate([meta2, meta1], axis=0)
    gathered = jnp.dot(perm, combined, preferred_element_type=jnp.float32)
    msel = jnp.sum(perm * vals, axis=1, keepdims=True)

    mean2 = jnp.sum(meta2, axis=1, keepdims=True) / C
    logits = jnp.dot(w, mean2, preferred_element_type=jnp.float32) + b
    logits = logits - jnp.max(logits)
    e = jnp.exp(logits)
    modu = e / jnp.sum(e)
    sel = (jax.lax.broadcasted_iota(jnp.int32, (M, K2), 1).astype(jnp.float32)
           == msel).astype(jnp.float32)
    scale = jnp.sum(sel * modu.T, axis=1, keepdims=True)
    return gathered * scale


def _body(x_ref, n1_ref, n2_ref, w_ref, b_ref, out_ref, *, k1, k2, K1, K2, F):
    xs = [x_ref[f] for f in range(F)]
    noise1 = jnp.concatenate([n1_ref[f] for f in range(F)], axis=1)
    noise2 = jnp.concatenate([n2_ref[f] for f in range(F)], axis=1)
    idx1, meta1 = _batched_cluster(xs, noise1, k1, K1)
    del idx1
    idx2, meta2 = _batched_cluster(meta1, noise2, k2, K2)
    for f in range(F):
        out_ref[f] = _group(meta1[f], meta2[f], idx2[f], w_ref[...], b_ref[...])


def kernel(vis_embed, score_w, score_b):
    vis_embed = vis_embed.astype(jnp.float32)
    T, N, C = vis_embed.shape
    K1 = max(math.ceil(N * 0.0625), 1)
    K2 = max(math.ceil(K1 * 0.35), 1)
    noise1 = (jax.random.uniform(jax.random.key(1), (T, N), dtype=jnp.float32)
              * 1e-6).reshape(T, 1, N)
    noise2 = (jax.random.uniform(jax.random.key(2), (T, K1), dtype=jnp.float32)
              * 1e-6).reshape(T, 1, K1)
    F = T
    out = pl.pallas_call(
        functools.partial(_body, k1=8, k2=3, K1=K1, K2=K2, F=F),
        grid=(T // F,),
        in_specs=[
            pl.BlockSpec((F, N, C), lambda t: (t, 0, 0)),
            pl.BlockSpec((F, 1, N), lambda t: (t, 0, 0)),
            pl.BlockSpec((F, 1, K1), lambda t: (t, 0, 0)),
            pl.BlockSpec((K2, K2), lambda t: (0, 0)),
            pl.BlockSpec((K2, 1), lambda t: (0, 0)),
        ],
        out_specs=pl.BlockSpec((F, K1 + K2, C), lambda t: (t, 0, 0)),
        out_shape=jax.ShapeDtypeStruct((T, K1 + K2, C), jnp.float32),
    )(vis_embed, noise1, noise2,
      score_w.astype(jnp.float32), score_b.astype(jnp.float32).reshape(K2, 1))
    return out


# speedup vs baseline: 1.1598x; 1.1598x over previous
import functools
import math

import jax
import jax.numpy as jnp
from jax.experimental import pallas as pl

_BIG = 3.0e38


def _batched_cluster(xs, noise_row, k, n_clusters):
    F = len(xs)
    N, C = xs[0].shape
    s = jnp.float32(math.sqrt(C))
    d2s = []
    dmaxs = []
    for x in xs:
        x2 = jnp.sum(x * x, axis=1, keepdims=True)
        dot = jnp.dot(x, x.T, preferred_element_type=jnp.float32)
        d2 = jnp.maximum((x2 + x2.T) - 2.0 * dot, 0.0)
        d2s.append(d2)
        dmaxs.append(jnp.max(d2))
    D = jnp.concatenate(d2s, axis=1)
    del d2s

    acc = jnp.zeros((1, F * N), jnp.float32)
    taken = jnp.zeros((1, F * N), jnp.float32)
    dmm = D
    for r in range(k):
        minv = jnp.min(dmm, axis=0, keepdims=True)
        t = jnp.sqrt(minv) / s
        eq = dmm == minv
        c = jnp.sum(eq.astype(jnp.float32), axis=0, keepdims=True)
        m = jnp.minimum(c, k - taken)
        acc = acc + m * (t * t)
        taken = taken + m
        if r < k - 1:
            dmm = jnp.where(eq, _BIG, dmm)
    density = jnp.exp(-(acc / k)) + noise_row

    dists = []
    for f in range(F):
        drow = density[:, f * N:(f + 1) * N]
        m = jnp.where(drow.T > drow, D[:, f * N:(f + 1) * N], dmaxs[f])
        dists.append(jnp.min(m, axis=0, keepdims=True))
    dist = jnp.sqrt(jnp.concatenate(dists, axis=1)) / s
    score = dist * density

    io0 = jax.lax.broadcasted_iota(jnp.int32, (N, N), 0)
    io1 = jax.lax.broadcasted_iota(jnp.int32, (N, N), 1)
    lower = io0 < io1
    idx_rows = []
    merged = []
    for f in range(F):
        Dv = D[:, f * N:(f + 1) * N]
        sr = score[:, f * N:(f + 1) * N]
        sc = sr.T
        beats = (sc > sr) | ((sc == sr) & lower)
        rank = jnp.sum(beats.astype(jnp.float32), axis=0, keepdims=True)
        rank_col = rank.T
        icen_col = rank_col < n_clusters
        masked = jnp.where(icen_col, Dv, _BIG)
        mv = jnp.min(masked, axis=0, keepdims=True)
        rc = jnp.where(icen_col, rank_col, jnp.float32(n_clusters))
        cl = jnp.min(jnp.where(masked == mv, rc, jnp.float32(n_clusters)),
                     axis=0, keepdims=True)
        idx_row = jnp.where(rank < n_clusters, rank, cl)
        assign = (jax.lax.broadcasted_iota(
            jnp.int32, (n_clusters, N), 0).astype(jnp.float32)
            == idx_row).astype(jnp.float32)
        counts = jnp.sum(assign, axis=1, keepdims=True)
        mrg = jnp.dot(assign, xs[f], preferred_element_type=jnp.float32)
        merged.append(mrg / (counts + 1e-6))
        idx_rows.append(idx_row)
    return idx_rows, merged


def _group(meta1, meta2, idx2_row, w, b):
    K1, C = meta1.shape
    K2 = meta2.shape[0]
    M = K1 + K2
    i_m = jax.lax.broadcasted_iota(jnp.int32, (1, M), 1).astype(jnp.float32)
    vals = jnp.concatenate(
        [jax.lax.broadcasted_iota(jnp.int32, (1, K2), 1).astype(jnp.float32),
         idx2_row], axis=1)
    rank = jnp.where(i_m < K2, 0.0, i_m - jnp.float32(K2 - 1))
    keys = vals * jnp.float32(K1 + 1) + rank
    pos = jnp.sum((keys < keys.T).astype(jnp.float32), axis=1, keepdims=True)
    perm = (jax.lax.broadcasted_iota(jnp.int32, (M, M), 0).astype(jnp.float32)
            == pos.T).astype(jnp.float32)
    combined = jnp.concatenate([meta2, meta1], axis=0)
    gathered = jnp.dot(perm, combined, preferred_element_type=jnp.float32)
    msel = jnp.sum(perm * vals, axis=1, keepdims=True)

    mean2 = jnp.sum(meta2, axis=1, keepdims=True) / C
    logits = jnp.dot(w, mean2, preferred_element_type=jnp.float32) + b
    logits = logits - jnp.max(logits)
    e = jnp.exp(logits)
    modu = e / jnp.sum(e)
    sel = (jax.lax.broadcasted_iota(jnp.int32, (M, K2), 1).astype(jnp.float32)
           == msel).astype(jnp.float32)
    scale = jnp.sum(sel * modu.T, axis=1, keepdims=True)
    return gathered * scale


def _body(x_ref, n1_ref, n2_ref, w_ref, b_ref, out_ref, *, k1, k2, K1, K2, F):
    xs = [x_ref[f] for f in range(F)]
    noise1 = jnp.concatenate([n1_ref[f] for f in range(F)], axis=1)
    noise2 = jnp.concatenate([n2_ref[f] for f in range(F)], axis=1)
    idx1, meta1 = _batched_cluster(xs, noise1, k1, K1)
    del idx1
    idx2, meta2 = _batched_cluster(meta1, noise2, k2, K2)
    for f in range(F):
        out_ref[f] = _group(meta1[f], meta2[f], idx2[f], w_ref[...], b_ref[...])


def kernel(vis_embed, score_w, score_b):
    vis_embed = vis_embed.astype(jnp.float32)
    T, N, C = vis_embed.shape
    K1 = max(math.ceil(N * 0.0625), 1)
    K2 = max(math.ceil(K1 * 0.35), 1)
    noise1 = (jax.random.uniform(jax.random.key(1), (T, N), dtype=jnp.float32)
              * 1e-6).reshape(T, 1, N)
    noise2 = (jax.random.uniform(jax.random.key(2), (T, K1), dtype=jnp.float32)
              * 1e-6).reshape(T, 1, K1)
    F = 8
    out = pl.pallas_call(
        functools.partial(_body, k1=8, k2=3, K1=K1, K2=K2, F=F),
        grid=(T // F,),
        in_specs=[
            pl.BlockSpec((F, N, C), lambda t: (t, 0, 0)),
            pl.BlockSpec((F, 1, N), lambda t: (t, 0, 0)),
            pl.BlockSpec((F, 1, K1), lambda t: (t, 0, 0)),
            pl.BlockSpec((K2, K2), lambda t: (0, 0)),
            pl.BlockSpec((K2, 1), lambda t: (0, 0)),
        ],
        out_specs=pl.BlockSpec((F, K1 + K2, C), lambda t: (t, 0, 0)),
        out_shape=jax.ShapeDtypeStruct((T, K1 + K2, C), jnp.float32),
    )(vis_embed, noise1, noise2,
      score_w.astype(jnp.float32), score_b.astype(jnp.float32).reshape(K2, 1))
    return out
